# baseline (speedup 1.0000x reference)
import functools

import jax
import jax.numpy as jnp
from jax import lax
from jax.experimental import pallas as pl
from jax.experimental.pallas import tpu as pltpu
from jax.experimental.pallas import tpu_sc as plsc

D_MODEL = 64
SCALE = 8.0

NC = 2
NS = 16
LANES = 16
NW = NC * NS

VOCAB_MAIN = 999936
N_GROUPS = VOCAB_MAIN // 128

SUP = 1024
GCH = 128
NG = SUP // GCH


@functools.lru_cache(maxsize=None)
def _make_detile(vocab):
    n_tail = vocab - VOCAB_MAIN
    mesh = plsc.VectorSubcoreMesh(
        core_axis_name="c", subcore_axis_name="s",
        num_cores=NC, num_subcores=NS)
    base_groups = N_GROUPS // NW
    extra = N_GROUPS - base_groups * NW

    @functools.partial(
        pl.kernel,
        mesh=mesh,
        out_type=jax.ShapeDtypeStruct((vocab * D_MODEL,), jnp.float32),
        scratch_types=[
            pltpu.VMEM((8, 8, 128), jnp.float32),
            pltpu.VMEM((128 * D_MODEL,), jnp.float32),
            pltpu.VMEM((n_tail * D_MODEL,), jnp.float32),
            pltpu.SemaphoreType.DMA,
        ],
        compiler_params=pltpu.CompilerParams(
            use_tc_tiling_on_sc=True, needs_layout_passes=False),
    )
    def detile(tt_hbm, tail_hbm, out_hbm, tiles_v, rows_v, tail_v, sem):
        wid = lax.axis_index("s") * NC + lax.axis_index("c")
        base_g = wid * base_groups + jnp.minimum(wid, extra)
        n_g = base_groups + jnp.where(wid < extra, 1, 0)

        def group_body(g, carry):
            gid = base_g + g
            copies = [
                pltpu.async_copy(
                    tt_hbm.at[pl.ds(jh * 8, 8), pl.ds(gid * 128, 128)],
                    tiles_v.at[jh], sem)
                for jh in range(8)
            ]
            for cp in copies:
                cp.wait()

            lane = lax.iota(jnp.int32, LANES)

            def ch_body(j, c2):
                jh = j // 8
                jl = j % 8
                for il in range(8):
                    val = tiles_v[jh, jl, pl.ds(il * LANES, LANES)]
                    plsc.store_scatter(
                        rows_v,
                        [lane * D_MODEL + (il * LANES * D_MODEL + j)],
                        val * SCALE)
                return c2

            lax.fori_loop(0, D_MODEL, ch_body, 0)
            pltpu.sync_copy(
                rows_v, out_hbm.at[pl.ds(gid * 128 * D_MODEL, 128 * D_MODEL)])
            return carry

        lax.fori_loop(0, n_g, group_body, 0)

        @pl.when(wid == NW - 1)
        def _():
            pltpu.sync_copy(tail_hbm, tail_v)

            def tail_scale(i, c2):
                tail_v[pl.ds(i * LANES, LANES)] = (
                    tail_v[pl.ds(i * LANES, LANES)] * SCALE)
                return c2

            lax.fori_loop(0, n_tail * D_MODEL // LANES, tail_scale, 0)
            pltpu.sync_copy(
                tail_v,
                out_hbm.at[pl.ds(VOCAB_MAIN * D_MODEL, n_tail * D_MODEL)])

    return detile


@functools.lru_cache(maxsize=None)
def _make_lookup(b, l, vocab):
    n_b = b // 128
    tok_w = 128 * l
    mesh = plsc.VectorSubcoreMesh(
        core_axis_name="c", subcore_axis_name="s",
        num_cores=NC, num_subcores=NS)

    @functools.partial(
        pl.kernel,
        mesh=mesh,
        out_type=jax.ShapeDtypeStruct(
            (l, D_MODEL // 8, n_b, 1024), jnp.float32),
        scratch_types=[
            pltpu.VMEM((tok_w,), jnp.int32),
            pltpu.VMEM((l, 128), jnp.int32),
            pltpu.VMEM((128, D_MODEL), jnp.float32),
            pltpu.VMEM((128 * D_MODEL,), jnp.float32),
            pltpu.SemaphoreType.DMA,
            pltpu.SemaphoreType.DMA,
        ],
        compiler_params=pltpu.CompilerParams(
            use_tc_tiling_on_sc=False, needs_layout_passes=False),
    )
    def lookup(table_hbm, idx_hbm, out_hbm, xblk, idxT, rows_v, tiles_v,
               sem_g, sem_w):
        wid = lax.axis_index("s") * NC + lax.axis_index("c")
        pltpu.sync_copy(idx_hbm.at[pl.ds(wid * tok_w, tok_w)], xblk)

        lane = lax.iota(jnp.int32, LANES)
        lane_l = lane * l

        def rbody(r, c2):
            for bl16 in range(8):
                val = plsc.load_gather(xblk, [lane_l + (bl16 * 16 * l + r)])
                idxT[r, pl.ds(bl16 * 16, 16)] = val
            return c2

        lax.fori_loop(0, l, rbody, 0)

        base_sc = (lane // 8) * 1024 + (lane % 8) * 128

        def drain_writes():
            for jh in range(D_MODEL // 8):
                pltpu.make_async_copy(
                    out_hbm.at[0, 0, 0],
                    tiles_v.at[pl.ds(jh * 1024, 1024)], sem_w).wait()

        def gather_r(r, c2):
            cp = pltpu.async_copy(table_hbm.at[idxT.at[r]], rows_v, sem_g)

            @pl.when(r > 0)
            def _():
                drain_writes()

            cp.wait()
            for j16 in range(D_MODEL // 16):
                for bl in range(128):
                    plsc.store_scatter(
                        tiles_v,
                        [base_sc + (j16 * 2048 + bl)],
                        rows_v[bl, pl.ds(j16 * 16, 16)])
            for jh in range(D_MODEL // 8):
                pltpu.async_copy(
                    tiles_v.at[pl.ds(jh * 1024, 1024)],
                    out_hbm.at[r, jh, wid], sem_w)
            return c2

        lax.fori_loop(0, l, gather_r, 0)
        drain_writes()

    return lookup


def kernel(x, table):
    b, l = x.shape
    vocab = table.shape[0]
    idx = x.reshape(b * l).astype(jnp.int32)
    tail = table[VOCAB_MAIN:].reshape((vocab - VOCAB_MAIN) * D_MODEL)
    flat = _make_detile(vocab)(table.T, tail)
    table_lin = flat.reshape(vocab, D_MODEL)
    out4 = _make_lookup(b, l, vocab)(table_lin, idx)
    out5 = out4.reshape(l, D_MODEL // 8, b // 128, 8, 128)
    return out5.transpose((2, 4, 0, 1, 3)).reshape(b, l, D_MODEL)


# speedup vs baseline: 1.0216x; 1.0216x over previous
import functools

import jax
import jax.numpy as jnp
from jax import lax
from jax.experimental import pallas as pl
from jax.experimental.pallas import tpu as pltpu
from jax.experimental.pallas import tpu_sc as plsc

D_MODEL = 64
SCALE = 8.0

NC = 2
NS = 16
LANES = 16
NW = NC * NS

VOCAB_MAIN = 999936
N_GROUPS = VOCAB_MAIN // 128


@functools.lru_cache(maxsize=None)
def _make_detile(vocab):
    n_tail = vocab - VOCAB_MAIN
    mesh = plsc.VectorSubcoreMesh(
        core_axis_name="c", subcore_axis_name="s",
        num_cores=NC, num_subcores=NS)
    base_groups = N_GROUPS // NW
    extra = N_GROUPS - base_groups * NW

    @functools.partial(
        pl.kernel,
        mesh=mesh,
        out_type=jax.ShapeDtypeStruct((vocab * D_MODEL,), jnp.float32),
        scratch_types=[
            pltpu.VMEM((D_MODEL, 128), jnp.float32),
            pltpu.VMEM((128 * D_MODEL,), jnp.float32),
            pltpu.VMEM((n_tail * D_MODEL,), jnp.float32),
            pltpu.SemaphoreType.DMA,
            pltpu.SemaphoreType.DMA,
        ],
        compiler_params=pltpu.CompilerParams(
            use_tc_tiling_on_sc=True, needs_layout_passes=False,
            disable_bounds_checks=True),
    )
    def detile(tt_hbm, tail_hbm, out_hbm, slab_v, rows_v, tail_v,
               sem_l, sem_w):
        wid = lax.axis_index("s") * NC + lax.axis_index("c")
        base_g = wid * base_groups + jnp.minimum(wid, extra)
        n_g = base_groups + jnp.where(wid < extra, 1, 0)

        lane = lax.iota(jnp.int32, LANES)
        lane_d = lane * D_MODEL

        def group_body(g, carry):
            gid = base_g + g
            cp = pltpu.async_copy(
                tt_hbm.at[pl.ds(0, D_MODEL), pl.ds(gid * 128, 128)],
                slab_v, sem_l)

            @pl.when(g > 0)
            def _():
                pltpu.make_async_copy(
                    out_hbm.at[pl.ds(0, 128 * D_MODEL)], rows_v,
                    sem_w).wait()

            cp.wait()
            for il in range(8):
                for j in range(D_MODEL):
                    plsc.store_scatter(
                        rows_v,
                        [lane_d + (il * LANES * D_MODEL + j)],
                        slab_v[j, pl.ds(il * LANES, LANES)] * SCALE)
            pltpu.async_copy(
                rows_v,
                out_hbm.at[pl.ds(gid * 128 * D_MODEL, 128 * D_MODEL)],
                sem_w)
            return carry

        lax.fori_loop(0, n_g, group_body, 0)
        pltpu.make_async_copy(
            out_hbm.at[pl.ds(0, 128 * D_MODEL)], rows_v, sem_w).wait()

        @pl.when(wid == NW - 1)
        def _():
            pltpu.sync_copy(tail_hbm, tail_v)

            def tail_scale(i, c2):
                tail_v[pl.ds(i * LANES, LANES)] = (
                    tail_v[pl.ds(i * LANES, LANES)] * SCALE)
                return c2

            lax.fori_loop(0, n_tail * D_MODEL // LANES, tail_scale, 0)
            pltpu.sync_copy(
                tail_v,
                out_hbm.at[pl.ds(VOCAB_MAIN * D_MODEL, n_tail * D_MODEL)])

    return detile


@functools.lru_cache(maxsize=None)
def _make_lookup(b, l, vocab):
    n_b = b // 128
    tok_w = 128 * l
    mesh = plsc.VectorSubcoreMesh(
        core_axis_name="c", subcore_axis_name="s",
        num_cores=NC, num_subcores=NS)

    @functools.partial(
        pl.kernel,
        mesh=mesh,
        out_type=jax.ShapeDtypeStruct(
            (l, D_MODEL // 8, n_b, 1024), jnp.float32),
        scratch_types=[
            pltpu.VMEM((tok_w,), jnp.int32),
            pltpu.VMEM((l, 128), jnp.int32),
            pltpu.VMEM((128, D_MODEL), jnp.float32),
            pltpu.VMEM((128 * D_MODEL,), jnp.float32),
            pltpu.SemaphoreType.DMA,
            pltpu.SemaphoreType.DMA,
        ],
        compiler_params=pltpu.CompilerParams(
            use_tc_tiling_on_sc=False, needs_layout_passes=False,
            disable_bounds_checks=True),
    )
    def lookup(table_hbm, idx_hbm, out_hbm, xblk, idxT, rows_v, tiles_v,
               sem_g, sem_w):
        wid = lax.axis_index("s") * NC + lax.axis_index("c")
        pltpu.sync_copy(idx_hbm.at[pl.ds(wid * tok_w, tok_w)], xblk)

        lane = lax.iota(jnp.int32, LANES)
        lane_l = lane * l

        def rbody(r, c2):
            for bl16 in range(8):
                val = plsc.load_gather(xblk, [lane_l + (bl16 * 16 * l + r)])
                idxT[r, pl.ds(bl16 * 16, 16)] = val
            return c2

        lax.fori_loop(0, l, rbody, 0)

        base_sc = (lane // 8) * 1024 + (lane % 8) * 128

        def gather_r(r, c2):
            cp = pltpu.async_copy(table_hbm.at[idxT.at[r]], rows_v, sem_g)

            @pl.when(r > 0)
            def _():
                for jh in range(D_MODEL // 8):
                    pltpu.make_async_copy(
                        out_hbm.at[0, 0, 0],
                        tiles_v.at[pl.ds(jh * 1024, 1024)], sem_w).wait()

            cp.wait()
            for j16 in range(D_MODEL // 16):
                for bl in range(128):
                    plsc.store_scatter(
                        tiles_v,
                        [base_sc + (j16 * 2048 + bl)],
                        rows_v[bl, pl.ds(j16 * 16, 16)])
            for jh in range(D_MODEL // 8):
                pltpu.async_copy(
                    tiles_v.at[pl.ds(jh * 1024, 1024)],
                    out_hbm.at[r, jh, wid], sem_w)
            return c2

        lax.fori_loop(0, l, gather_r, 0)
        for jh in range(D_MODEL // 8):
            pltpu.make_async_copy(
                out_hbm.at[0, 0, 0],
                tiles_v.at[pl.ds(jh * 1024, 1024)], sem_w).wait()

    return lookup


def kernel(x, table):
    b, l = x.shape
    vocab = table.shape[0]
    idx = x.reshape(b * l).astype(jnp.int32)
    tail = table[VOCAB_MAIN:].reshape((vocab - VOCAB_MAIN) * D_MODEL)
    flat = _make_detile(vocab)(table.T, tail)
    table_lin = flat.reshape(vocab, D_MODEL)
    out4 = _make_lookup(b, l, vocab)(table_lin, idx)
    out5 = out4.reshape(l, D_MODEL // 8, b // 128, 8, 128)
    return out5.transpose((2, 4, 0, 1, 3)).reshape(b, l, D_MODEL)


# speedup vs baseline: 1.0230x; 1.0014x over previous
import functools

import jax
import jax.numpy as jnp
from jax import lax
from jax.experimental import pallas as pl
from jax.experimental.pallas import tpu as pltpu
from jax.experimental.pallas import tpu_sc as plsc

D_MODEL = 64
SCALE = 8.0

NC = 2
NS = 16
LANES = 16
NW = NC * NS

VOCAB_MAIN = 999936
N_GROUPS = VOCAB_MAIN // 128


@functools.lru_cache(maxsize=None)
def _make_detile(vocab):
    n_tail = vocab - VOCAB_MAIN
    mesh = plsc.VectorSubcoreMesh(
        core_axis_name="c", subcore_axis_name="s",
        num_cores=NC, num_subcores=NS)
    base_groups = N_GROUPS // NW
    extra = N_GROUPS - base_groups * NW

    @functools.partial(
        pl.kernel,
        mesh=mesh,
        out_type=jax.ShapeDtypeStruct((vocab * D_MODEL,), jnp.float32),
        scratch_types=[
            pltpu.VMEM((D_MODEL, 128), jnp.float32),
            pltpu.VMEM((128 * D_MODEL,), jnp.float32),
            pltpu.VMEM((n_tail * D_MODEL,), jnp.float32),
            pltpu.SemaphoreType.DMA,
            pltpu.SemaphoreType.DMA,
        ],
        compiler_params=pltpu.CompilerParams(
            use_tc_tiling_on_sc=True, needs_layout_passes=False,
            disable_bounds_checks=True),
    )
    def detile(tt_hbm, tail_hbm, out_hbm, slab_v, rows_v, tail_v,
               sem_l, sem_w):
        wid = lax.axis_index("s") * NC + lax.axis_index("c")
        base_g = wid * base_groups + jnp.minimum(wid, extra)
        n_g = base_groups + jnp.where(wid < extra, 1, 0)

        lane = lax.iota(jnp.int32, LANES)
        lane_d = lane * D_MODEL

        def group_body(g, carry):
            gid = base_g + g
            cp = pltpu.async_copy(
                tt_hbm.at[pl.ds(0, D_MODEL), pl.ds(gid * 128, 128)],
                slab_v, sem_l)

            @pl.when(g > 0)
            def _():
                pltpu.make_async_copy(
                    out_hbm.at[pl.ds(0, 128 * D_MODEL)], rows_v,
                    sem_w).wait()

            cp.wait()
            idxv = lane_d
            for il in range(8):
                for j in range(D_MODEL):
                    plsc.store_scatter(
                        rows_v, [idxv],
                        slab_v[j, pl.ds(il * LANES, LANES)] * SCALE)
                    idxv = idxv + (1 if j < D_MODEL - 1
                                   else LANES * D_MODEL - D_MODEL + 1)
            pltpu.async_copy(
                rows_v,
                out_hbm.at[pl.ds(gid * 128 * D_MODEL, 128 * D_MODEL)],
                sem_w)
            return carry

        lax.fori_loop(0, n_g, group_body, 0)
        pltpu.make_async_copy(
            out_hbm.at[pl.ds(0, 128 * D_MODEL)], rows_v, sem_w).wait()

        @pl.when(wid == NW - 1)
        def _():
            pltpu.sync_copy(tail_hbm, tail_v)

            def tail_scale(i, c2):
                tail_v[pl.ds(i * LANES, LANES)] = (
                    tail_v[pl.ds(i * LANES, LANES)] * SCALE)
                return c2

            lax.fori_loop(0, n_tail * D_MODEL // LANES, tail_scale, 0)
            pltpu.sync_copy(
                tail_v,
                out_hbm.at[pl.ds(VOCAB_MAIN * D_MODEL, n_tail * D_MODEL)])

    return detile


@functools.lru_cache(maxsize=None)
def _make_lookup(b, l, vocab):
    n_b = b // 128
    tok_w = 128 * l
    mesh = plsc.VectorSubcoreMesh(
        core_axis_name="c", subcore_axis_name="s",
        num_cores=NC, num_subcores=NS)

    @functools.partial(
        pl.kernel,
        mesh=mesh,
        out_type=jax.ShapeDtypeStruct(
            (l, D_MODEL // 8, n_b, 1024), jnp.float32),
        scratch_types=[
            pltpu.VMEM((tok_w,), jnp.int32),
            pltpu.VMEM((l, 128), jnp.int32),
            pltpu.VMEM((128, D_MODEL), jnp.float32),
            pltpu.VMEM((128 * D_MODEL,), jnp.float32),
            pltpu.SemaphoreType.DMA,
            pltpu.SemaphoreType.DMA,
        ],
        compiler_params=pltpu.CompilerParams(
            use_tc_tiling_on_sc=False, needs_layout_passes=False,
            disable_bounds_checks=True),
    )
    def lookup(table_hbm, idx_hbm, out_hbm, xblk, idxT, rows_v, tiles_v,
               sem_g, sem_w):
        wid = lax.axis_index("s") * NC + lax.axis_index("c")
        pltpu.sync_copy(idx_hbm.at[pl.ds(wid * tok_w, tok_w)], xblk)

        lane = lax.iota(jnp.int32, LANES)
        lane_l = lane * l

        def rbody(r, idxr):
            iv = idxr
            for bl16 in range(8):
                val = plsc.load_gather(xblk, [iv])
                idxT[r, pl.ds(bl16 * 16, 16)] = val
                iv = iv + 16 * l
            return idxr + 1

        lax.fori_loop(0, l, rbody, lane_l)

        base_sc = (lane // 8) * 1024 + (lane % 8) * 128

        def gather_r(r, c2):
            cp = pltpu.async_copy(table_hbm.at[idxT.at[r]], rows_v, sem_g)

            @pl.when(r > 0)
            def _():
                for jh in range(D_MODEL // 8):
                    pltpu.make_async_copy(
                        out_hbm.at[0, 0, 0],
                        tiles_v.at[pl.ds(jh * 1024, 1024)], sem_w).wait()

            cp.wait()
            idxv = base_sc
            for j16 in range(D_MODEL // 16):
                for bl in range(128):
                    plsc.store_scatter(
                        tiles_v, [idxv],
                        rows_v[bl, pl.ds(j16 * 16, 16)])
                    idxv = idxv + (1 if bl < 127 else 2048 - 127)
            for jh in range(D_MODEL // 8):
                pltpu.async_copy(
                    tiles_v.at[pl.ds(jh * 1024, 1024)],
                    out_hbm.at[r, jh, wid], sem_w)
            return c2

        lax.fori_loop(0, l, gather_r, 0)
        for jh in range(D_MODEL // 8):
            pltpu.make_async_copy(
                out_hbm.at[0, 0, 0],
                tiles_v.at[pl.ds(jh * 1024, 1024)], sem_w).wait()

    return lookup


def kernel(x, table):
    b, l = x.shape
    vocab = table.shape[0]
    idx = x.reshape(b * l).astype(jnp.int32)
    tail = table[VOCAB_MAIN:].reshape((vocab - VOCAB_MAIN) * D_MODEL)
    flat = _make_detile(vocab)(table.T, tail)
    table_lin = flat.reshape(vocab, D_MODEL)
    out4 = _make_lookup(b, l, vocab)(table_lin, idx)
    out5 = out4.reshape(l, D_MODEL // 8, b // 128, 8, 128)
    return out5.transpose((2, 4, 0, 1, 3)).reshape(b, l, D_MODEL)


# speedup vs baseline: 2.1315x; 2.0837x over previous
import functools

import jax
import jax.numpy as jnp
from jax import lax
from jax.experimental import pallas as pl
from jax.experimental.pallas import tpu as pltpu
from jax.experimental.pallas import tpu_sc as plsc

D_MODEL = 64
SCALE = 8.0

NC = 2
NS = 16
LANES = 16
NW = NC * NS

SUP = 1024
GCH = 128
NG = SUP // GCH


@functools.lru_cache(maxsize=None)
def _make_lookup(n):
    b_per_w = n // NW
    n_sup = b_per_w // SUP
    mesh = plsc.VectorSubcoreMesh(
        core_axis_name="c", subcore_axis_name="s",
        num_cores=NC, num_subcores=NS)

    @functools.partial(
        pl.kernel,
        mesh=mesh,
        out_type=jax.ShapeDtypeStruct((n, D_MODEL), jnp.float32),
        scratch_types=[
            pltpu.VMEM((SUP,), jnp.int32),
            pltpu.VMEM((SUP, D_MODEL), jnp.float32),
            pltpu.SemaphoreType.DMA,
        ],
        compiler_params=pltpu.CompilerParams(use_tc_tiling_on_sc=False),
    )
    def lookup(table_hbm, idx_hbm, out_hbm, idx_v, rows_v, sem):
        wid = lax.axis_index("s") * NC + lax.axis_index("c")
        base = wid * b_per_w

        def sup_body(g, carry):
            off = base + g * SUP
            pltpu.sync_copy(idx_hbm.at[pl.ds(off, SUP)], idx_v)
            copies = [
                pltpu.async_copy(
                    table_hbm.at[idx_v.at[pl.ds(j * GCH, GCH)]],
                    rows_v.at[pl.ds(j * GCH, GCH)],
                    sem,
                )
                for j in range(NG)
            ]
            for cp in copies:
                cp.wait()

            def scale_row(r, c2):
                for c in range(D_MODEL // LANES):
                    rows_v[r, pl.ds(c * LANES, LANES)] = (
                        rows_v[r, pl.ds(c * LANES, LANES)] * SCALE)
                return c2

            lax.fori_loop(0, SUP, scale_row, 0)
            pltpu.sync_copy(rows_v, out_hbm.at[pl.ds(off, SUP)])
            return carry

        lax.fori_loop(0, n_sup, sup_body, 0)

    return lookup


def kernel(x, table):
    b, l = x.shape
    idx = x.reshape(b * l).astype(jnp.int32)
    out = _make_lookup(b * l)(table, idx)
    return out.reshape(b, l, D_MODEL)
